# baseline (speedup 1.0000x reference)
import functools

import jax
import jax.numpy as jnp
from jax import lax
from jax.experimental import pallas as pl
from jax.experimental.pallas import tpu as pltpu
from jax.experimental.pallas import tpu_sc as plsc

_B, _L, _D, _V = 4096, 50, 128, 100000
_NF = 3
_NC, _NS = 2, 16
_NW = _NC * _NS
_NSLICE = 4
_BSL = _B // _NSLICE
_BPW = _BSL // _NW
_NB = 2
_STEPS = _BPW // _NB
_RPB = _NF * _L
_RPAD = 152


def _make_sc_kernel():
    mesh = plsc.VectorSubcoreMesh(
        core_axis_name="c", subcore_axis_name="s",
        num_cores=_NC, num_subcores=_NS,
    )

    @functools.partial(
        pl.kernel,
        out_type=jax.ShapeDtypeStruct((_BSL, _RPAD, _D), jnp.float32),
        mesh=mesh,
        scratch_types=[
            pltpu.VMEM((_NF, _BPW, _L), jnp.int32),
            pltpu.VMEM((2, _NB, _RPAD, _D), jnp.float32),
            pltpu.SemaphoreType.DMA,
            pltpu.SemaphoreType.DMA,
            pltpu.SemaphoreType.DMA,
            pltpu.SemaphoreType.DMA,
        ],
    )
    def emb(v_e, v_i, v_c, w_e, w_i, w_c, out, idx_v, rows_v,
            sem_g0, sem_g1, sem_w0, sem_w1):
        wid = lax.axis_index("s") * _NC + lax.axis_index("c")
        b_base = wid * _BPW
        sems = (sem_g0, sem_g1)
        wsems = (sem_w0, sem_w1)
        tables = (w_e, w_i, w_c)

        pltpu.sync_copy(v_e.at[pl.ds(b_base, _BPW)], idx_v.at[0])
        pltpu.sync_copy(v_i.at[pl.ds(b_base, _BPW)], idx_v.at[1])
        pltpu.sync_copy(v_c.at[pl.ds(b_base, _BPW)], idx_v.at[2])

        def fire(s, k):
            for bl in range(_NB):
                for f in range(_NF):
                    dst = rows_v.at[k, bl, pl.ds(f * _L, _L)]
                    pltpu.async_copy(
                        tables[f].at[idx_v.at[f, s * _NB + bl]], dst, sems[k]
                    )
                pltpu.async_copy(
                    tables[0].at[idx_v.at[0, s * _NB + bl, pl.ds(0, 2)]],
                    rows_v.at[k, bl, pl.ds(_RPB, _RPAD - _RPB)],
                    sems[k],
                )

        def drain(k):
            pltpu.make_async_copy(
                out.at[pl.ds(0, _NB)], rows_v.at[k], sems[k]
            ).wait()

        def fire_write(s, k):
            row_b = b_base + s * _NB
            pltpu.async_copy(rows_v.at[k], out.at[pl.ds(row_b, _NB)], wsems[k])

        def drain_write(k):
            pltpu.make_async_copy(
                rows_v.at[k], out.at[pl.ds(0, _NB)], wsems[k]
            ).wait()

        fire(0, 0)

        def body(i, _):
            for k in range(2):
                s = 2 * i + k
                drain(k)
                fire_write(s, k)
                if k == 0:
                    @pl.when(i != 0)
                    def _():
                        drain_write(1)
                    fire(s + 1, 1)
                else:
                    drain_write(0)
                    @pl.when(i != _STEPS // 2 - 1)
                    def _():
                        fire(s + 1, 0)
            return ()

        lax.fori_loop(0, _STEPS // 2, body, ())
        drain_write(1)

    return emb


_emb = _make_sc_kernel()


def kernel(v_event, v_item, v_category, W_event, W_item, W_category):
    full = jnp.zeros((_B, _RPB, _D), jnp.float32)
    for i in range(_NSLICE):
        sl = slice(i * _BSL, (i + 1) * _BSL)
        part = _emb(v_event[sl], v_item[sl], v_category[sl],
                    W_event, W_item, W_category)
        full = lax.dynamic_update_slice(
            full, part[:, :_RPB, :], (i * _BSL, 0, 0)
        )
    return full


# speedup vs baseline: 1.7950x; 1.7950x over previous
import functools

import jax
import jax.numpy as jnp
from jax import lax
from jax.experimental import pallas as pl
from jax.experimental.pallas import tpu as pltpu
from jax.experimental.pallas import tpu_sc as plsc

_B, _L, _D, _V = 4096, 50, 128, 100000
_NF = 3
_NC, _NS = 2, 16
_NW = _NC * _NS
_BPW = _B // _NW
_NB = 1
_NBUF = 4
_STEPS = _BPW // _NB
_RPB = _NF * _L


def _make_kernel():
    mesh = plsc.VectorSubcoreMesh(
        core_axis_name="c", subcore_axis_name="s",
        num_cores=_NC, num_subcores=_NS,
    )

    @functools.partial(
        pl.kernel,
        out_type=jax.ShapeDtypeStruct((_B, _RPB, _D), jnp.float32),
        mesh=mesh,
        scratch_types=[
            pltpu.VMEM((_NF, _BPW, _L), jnp.int32),
            pltpu.VMEM((_NBUF, _NB, _RPB, _D), jnp.float32),
            pltpu.SemaphoreType.DMA,
            pltpu.SemaphoreType.DMA,
            pltpu.SemaphoreType.DMA,
            pltpu.SemaphoreType.DMA,
            pltpu.SemaphoreType.DMA,
            pltpu.SemaphoreType.DMA,
            pltpu.SemaphoreType.DMA,
            pltpu.SemaphoreType.DMA,
        ],
    )
    def emb(v_e, v_i, v_c, w_e, w_i, w_c, out, idx_v, rows_v,
            sem_g0, sem_g1, sem_g2, sem_g3, sem_w0, sem_w1, sem_w2, sem_w3):
        wid = lax.axis_index("s") * _NC + lax.axis_index("c")
        b_base = wid * _BPW
        sems = (sem_g0, sem_g1, sem_g2, sem_g3)
        wsems = (sem_w0, sem_w1, sem_w2, sem_w3)
        tables = (w_e, w_i, w_c)

        pltpu.sync_copy(v_e.at[pl.ds(b_base, _BPW)], idx_v.at[0])
        pltpu.sync_copy(v_i.at[pl.ds(b_base, _BPW)], idx_v.at[1])
        pltpu.sync_copy(v_c.at[pl.ds(b_base, _BPW)], idx_v.at[2])

        def fire(s, k):
            for bl in range(_NB):
                for f in range(_NF):
                    dst = rows_v.at[k, bl, pl.ds(f * _L, _L)]
                    pltpu.async_copy(
                        tables[f].at[idx_v.at[f, s * _NB + bl]], dst, sems[k]
                    )

        def drain(k):
            pltpu.make_async_copy(
                out.at[pl.ds(0, _NB)], rows_v.at[k], sems[k]
            ).wait()

        def fire_write(s, k):
            row_b = b_base + s * _NB
            pltpu.async_copy(rows_v.at[k], out.at[pl.ds(row_b, _NB)], wsems[k])

        def drain_write(k):
            pltpu.make_async_copy(
                rows_v.at[k], out.at[pl.ds(0, _NB)], wsems[k]
            ).wait()

        for p in range(_NBUF - 1):
            fire(p, p)

        def body(i, _):
            for k in range(_NBUF):
                s = _NBUF * i + k
                kn = (k + _NBUF - 1) % _NBUF
                drain(k)
                fire_write(s, k)
                if k == 0:
                    @pl.when(i != 0)
                    def _():
                        drain_write(kn)
                    fire(s + _NBUF - 1, kn)
                else:
                    drain_write(kn)
                    @pl.when(i != _STEPS // _NBUF - 1)
                    def _():
                        fire(s + _NBUF - 1, kn)
            return ()

        lax.fori_loop(0, _STEPS // _NBUF, body, ())
        drain_write((_STEPS - 1) % _NBUF)

    return emb


_emb = _make_kernel()


def kernel(v_event, v_item, v_category, W_event, W_item, W_category):
    return _emb(v_event, v_item, v_category, W_event, W_item, W_category)
